# baseline (speedup 1.0000x reference)
import jax
import jax.numpy as jnp
from jax.experimental import pallas as pl

_BM = 64


def _fused_kernel(x_ref, emb_ref, o_ref):
    x = jnp.log1p(x_ref[...]).astype(jnp.bfloat16)
    o_ref[...] = jnp.dot(x, emb_ref[...], preferred_element_type=jnp.float32)


def kernel(x_seq, bb_gene_emb):
    m, k = x_seq.shape
    _, n = bb_gene_emb.shape
    emb_bf = bb_gene_emb.astype(jnp.bfloat16)
    return pl.pallas_call(
        _fused_kernel,
        grid=(m // _BM,),
        in_specs=[
            pl.BlockSpec((_BM, k), lambda i: (i, 0)),
            pl.BlockSpec((k, n), lambda i: (0, 0)),
        ],
        out_specs=pl.BlockSpec((_BM, n), lambda i: (i, 0)),
        out_shape=jax.ShapeDtypeStruct((m, n), jnp.float32),
    )(x_seq, emb_bf)


# speedup vs baseline: 1.3705x; 1.3705x over previous
import jax
import jax.numpy as jnp
from jax.experimental import pallas as pl

_K = 19264
_BK = 512
_NSTEPS = 38
_TAIL0 = 256


def _fused_kernel(x_ref, emb_ref, o_ref):
    j = pl.program_id(0)

    @pl.when(j == 0)
    def _init():
        o_ref[...] = jnp.zeros_like(o_ref)

    @pl.when(j == _NSTEPS - 1)
    def _sanitize_tail():
        tail = x_ref[:, _TAIL0:]
        x_ref[:, _TAIL0:] = jnp.where((tail > 0) & (tail < 1e30), tail, 0.0)

    x = jnp.log1p(x_ref[...]).astype(jnp.bfloat16)
    o_ref[...] += jnp.dot(x, emb_ref[...], preferred_element_type=jnp.float32)


def kernel(x_seq, bb_gene_emb):
    m, k = x_seq.shape
    _, n = bb_gene_emb.shape
    emb_bf = jnp.pad(
        bb_gene_emb.astype(jnp.bfloat16), ((0, _NSTEPS * _BK - k), (0, 0))
    )
    return pl.pallas_call(
        _fused_kernel,
        grid=(_NSTEPS,),
        in_specs=[
            pl.BlockSpec((m, _BK), lambda j: (0, j)),
            pl.BlockSpec((_BK, n), lambda j: (j, 0)),
        ],
        out_specs=pl.BlockSpec((m, n), lambda j: (0, 0)),
        out_shape=jax.ShapeDtypeStruct((m, n), jnp.float32),
    )(x_seq, emb_bf)


# speedup vs baseline: 1.4713x; 1.0736x over previous
import jax
import jax.numpy as jnp
from jax.experimental import pallas as pl

_K = 19264
_BM = 2048
_BK = 1024
_NSTEPS = 19
_TAIL0 = 768


def _fused_kernel(x_ref, emb_ref, o_ref):
    j = pl.program_id(1)

    @pl.when(j == 0)
    def _init():
        o_ref[...] = jnp.zeros_like(o_ref)

    @pl.when(j == _NSTEPS - 1)
    def _sanitize_tail():
        tail = x_ref[:, _TAIL0:]
        x_ref[:, _TAIL0:] = jnp.where((tail > 0) & (tail < 1e30), tail, 0.0)

    x = jnp.log(x_ref[...] + 1.0).astype(jnp.bfloat16)
    o_ref[...] += jnp.dot(x, emb_ref[...], preferred_element_type=jnp.float32)


def kernel(x_seq, bb_gene_emb):
    m, k = x_seq.shape
    _, n = bb_gene_emb.shape
    emb_bf = jnp.pad(
        bb_gene_emb.astype(jnp.bfloat16), ((0, _NSTEPS * _BK - k), (0, 0))
    )
    return pl.pallas_call(
        _fused_kernel,
        grid=(m // _BM, _NSTEPS),
        in_specs=[
            pl.BlockSpec((_BM, _BK), lambda i, j: (i, j)),
            pl.BlockSpec((_BK, n), lambda i, j: (j, 0)),
        ],
        out_specs=pl.BlockSpec((_BM, n), lambda i, j: (i, 0)),
        out_shape=jax.ShapeDtypeStruct((m, n), jnp.float32),
    )(x_seq, emb_bf)


# speedup vs baseline: 1.6504x; 1.1217x over previous
import jax
import jax.numpy as jnp
from jax.experimental import pallas as pl

_K = 19264
_BM = 2048
_BK = 1024
_NSTEPS = 19


def _fused_kernel(x_ref, emb_ref, o_ref):
    j = pl.program_id(1)

    @pl.when(j == 0)
    def _init():
        o_ref[...] = jnp.zeros_like(o_ref)

    valid = _K - j * _BK

    x = x_ref[...]
    lane = jax.lax.broadcasted_iota(jnp.int32, x.shape, 1)
    x = jnp.where(lane < valid, x, 0.0)
    y = jnp.log(x + 1.0).astype(jnp.bfloat16)

    e = emb_ref[...]
    row = jax.lax.broadcasted_iota(jnp.int32, e.shape, 0)
    e = jnp.where(row < valid, e, 0.0).astype(jnp.bfloat16)

    o_ref[...] += jnp.dot(y, e, preferred_element_type=jnp.float32)


def kernel(x_seq, bb_gene_emb):
    m, k = x_seq.shape
    _, n = bb_gene_emb.shape
    return pl.pallas_call(
        _fused_kernel,
        grid=(m // _BM, _NSTEPS),
        in_specs=[
            pl.BlockSpec((_BM, _BK), lambda i, j: (i, j)),
            pl.BlockSpec((_BK, n), lambda i, j: (j, 0)),
        ],
        out_specs=pl.BlockSpec((_BM, n), lambda i, j: (i, 0)),
        out_shape=jax.ShapeDtypeStruct((m, n), jnp.float32),
    )(x_seq, bb_gene_emb)


# speedup vs baseline: 3.9628x; 2.4012x over previous
import jax
import jax.numpy as jnp
from jax.experimental import pallas as pl

_K = 19264
_BM = 2048
_BK = 1024
_NSTEPS = 19


def _fused_kernel(xt_ref, emb_ref, o_ref):
    j = pl.program_id(1)

    @pl.when(j == 0)
    def _init():
        o_ref[...] = jnp.zeros_like(o_ref)

    valid = _K - j * _BK

    xt = xt_ref[...]
    row = jax.lax.broadcasted_iota(jnp.int32, xt.shape, 0)
    xt = jnp.where(row < valid, xt, 0.0)
    y = jnp.log(xt + 1.0).astype(jnp.bfloat16)

    e = emb_ref[...]
    erow = jax.lax.broadcasted_iota(jnp.int32, e.shape, 0)
    e = jnp.where(erow < valid, e, 0.0).astype(jnp.bfloat16)

    o_ref[...] += jax.lax.dot_general(
        y, e, (((0,), (0,)), ((), ())), preferred_element_type=jnp.float32
    )


def kernel(x_seq, bb_gene_emb):
    m, k = x_seq.shape
    _, n = bb_gene_emb.shape
    xt = x_seq.T
    return pl.pallas_call(
        _fused_kernel,
        grid=(m // _BM, _NSTEPS),
        in_specs=[
            pl.BlockSpec((_BK, _BM), lambda i, j: (j, i)),
            pl.BlockSpec((_BK, n), lambda i, j: (j, 0)),
        ],
        out_specs=pl.BlockSpec((_BM, n), lambda i, j: (i, 0)),
        out_shape=jax.ShapeDtypeStruct((m, n), jnp.float32),
    )(xt, bb_gene_emb)
